# baseline (speedup 1.0000x reference)
import functools

import jax
import jax.numpy as jnp
from jax import lax
from jax.experimental import pallas as pl
from jax.experimental.pallas import tpu as pltpu
from jax.experimental.pallas import tpu_sc as plsc


def _occ_sc_body(B, C, NCH, CP, added_hbm, out_hbm, rows_v, occ_v, sem):
    cid = lax.axis_index("c")
    sid = lax.axis_index("s")
    w = sid * 2 + cid

    def per_batch(b, carry):
        base = (b * C) * NCH + w
        idx = lax.iota(jnp.int32, 16) * NCH + base
        pltpu.async_copy(added_hbm.at[idx], rows_v, sem).wait()

        def per_vec(i, carry2):
            o = i * 16
            acc = rows_v[0, pl.ds(o, 16)]
            for c in range(1, C):
                acc = acc + rows_v[c, pl.ds(o, 16)]
            occ_v[pl.ds(o, 16)] = jnp.where(jnp.abs(acc) > 0.0, 1.0, 0.0)
            return carry2

        lax.fori_loop(0, CP // 16, per_vec, 0, unroll=4)
        pltpu.sync_copy(occ_v, out_hbm.at[b * NCH + w])
        return carry

    lax.fori_loop(0, B, per_batch, 0)


def _occ_sc(added_points):
    B, C, H, W = added_points.shape
    NCH = 32
    CP = (H * W) // NCH
    a2 = added_points.reshape(B * C * NCH, CP)

    mesh = plsc.VectorSubcoreMesh(core_axis_name="c", subcore_axis_name="s",
                                  num_cores=2)
    kern = pl.kernel(
        functools.partial(_occ_sc_body, B, C, NCH, CP),
        mesh=mesh,
        out_type=jax.ShapeDtypeStruct((B * NCH, CP), jnp.float32),
        scratch_types=[
            pltpu.VMEM((C, CP), jnp.float32),
            pltpu.VMEM((CP,), jnp.float32),
            pltpu.SemaphoreType.DMA,
        ],
    )
    return kern(a2).reshape(B, H, W)


def _cov_body(orig_ref, boxes_ref, boxesT_ref, m1_ref, m2_ref):
    H = orig_ref.shape[2]
    W = orig_ref.shape[3]

    orig = jnp.sum(orig_ref[0, 1:], axis=0)
    occ_o = jnp.abs(orig) > 0.0

    bT = boxesT_ref[0]
    bC = boxes_ref[0]
    M = bC.shape[0]

    c_r = jnp.cos(bT[6:7, :])
    s_r = jnp.sin(bT[6:7, :])
    zok_r = jnp.abs(bT[2:3, :]) < bT[5:6, :] * 0.5
    iex_r = 2.0 / bT[3:4, :]
    iey_r = 2.0 / bT[4:5, :]
    tx_r = jnp.where(zok_r, -(bT[0:1, :] * c_r + bT[1:2, :] * s_r) * iex_r, 1e9)
    ty_r = jnp.where(zok_r, (bT[0:1, :] * s_r - bT[1:2, :] * c_r) * iey_r, 1e9)

    c_c = jnp.cos(bC[:, 6:7])
    s_c = jnp.sin(bC[:, 6:7])
    iex_c = 2.0 / bC[:, 3:4]
    iey_c = 2.0 / bC[:, 4:5]

    xcol = lax.broadcasted_iota(jnp.int32, (H, 1), 0).astype(jnp.float32) * 0.8
    U1 = xcol * (c_r * iex_r) + tx_r
    U2 = xcol * (-s_r * iey_r) + ty_r

    yrow = lax.broadcasted_iota(jnp.int32, (1, W), 1).astype(jnp.float32) * 0.8
    V1 = (s_c * iex_c) * yrow
    V2 = (c_c * iey_c) * yrow

    score = None
    for m in range(M):
        lx = U1[:, m : m + 1] + V1[m : m + 1, :]
        ly = U2[:, m : m + 1] + V2[m : m + 1, :]
        d = jnp.maximum(jnp.abs(lx), jnp.abs(ly))
        score = d if score is None else jnp.minimum(score, d)
    in_any = score < 1.0

    m1_ref[0] = jnp.where(jnp.logical_and(in_any, occ_o), 1.0, 0.0)
    m2_ref[0] = jnp.where(in_any, 1.0, 0.0)


def _coverage_tc(original_points, boxes, boxesT):
    B, C1, H, W = original_points.shape
    M = boxes.shape[1]
    return pl.pallas_call(
        _cov_body,
        grid=(B,),
        in_specs=[
            pl.BlockSpec((1, C1, H, W), lambda b: (b, 0, 0, 0)),
            pl.BlockSpec((1, M, 7), lambda b: (b, 0, 0)),
            pl.BlockSpec((1, 7, M), lambda b: (b, 0, 0)),
        ],
        out_specs=[
            pl.BlockSpec((1, H, W), lambda b: (b, 0, 0)),
            pl.BlockSpec((1, H, W), lambda b: (b, 0, 0)),
        ],
        out_shape=[
            jax.ShapeDtypeStruct((B, H, W), jnp.float32),
            jax.ShapeDtypeStruct((B, H, W), jnp.float32),
        ],
        compiler_params=pltpu.CompilerParams(
            dimension_semantics=("arbitrary",)),
    )(original_points, boxes, boxesT)


def _comb_body(occp_ref, m1_ref, m2_ref, out_ref):
    H = occp_ref.shape[1]
    W = occp_ref.shape[2]
    occp = occp_ref[0]
    m1 = m1_ref[0]
    m2 = m2_ref[0]
    w_i = occp * m1
    w_u = jnp.maximum(m1, occp * m2)
    fi = jnp.zeros((8, 128), jnp.float32)
    fu = jnp.zeros((8, 128), jnp.float32)
    for r in range(H // 8):
        for cc in range(W // 128):
            fi = fi + w_i[8 * r : 8 * r + 8, 128 * cc : 128 * cc + 128]
            fu = fu + w_u[8 * r : 8 * r + 8, 128 * cc : 128 * cc + 128]
    inter = jnp.sum(fi)
    union = jnp.sum(fu)
    lane = lax.broadcasted_iota(jnp.int32, (1, 1, 128), 2)
    out_ref[...] = (jnp.where(lane == 0, inter, 0.0)
                    + jnp.where(lane == 1, union, 0.0))


def _combine_tc(occ_p, m1, m2):
    B, H, W = occ_p.shape
    return pl.pallas_call(
        _comb_body,
        grid=(B,),
        in_specs=[
            pl.BlockSpec((1, H, W), lambda b: (b, 0, 0)),
            pl.BlockSpec((1, H, W), lambda b: (b, 0, 0)),
            pl.BlockSpec((1, H, W), lambda b: (b, 0, 0)),
        ],
        out_specs=pl.BlockSpec((1, 1, 128), lambda b: (b, 0, 0)),
        out_shape=jax.ShapeDtypeStruct((B, 1, 128), jnp.float32),
        compiler_params=pltpu.CompilerParams(
            dimension_semantics=("arbitrary",)),
    )(occ_p, m1, m2)


def kernel(added_points, original_points, boxes):
    M = boxes.shape[1]
    boxesT = jnp.transpose(boxes, (0, 2, 1))

    occ_p = _occ_sc(added_points)
    m1, m2 = _coverage_tc(original_points, boxes, boxesT)
    out = _combine_tc(occ_p, m1, m2)

    inter = out[:, 0, 0]
    union = out[:, 0, 1]
    return jnp.mean(M * inter / (union + 1e-6))


# speedup vs baseline: 3.3914x; 3.3914x over previous
import jax
import jax.numpy as jnp
from jax import lax
from jax.experimental import pallas as pl
from jax.experimental.pallas import tpu as pltpu


def _body(added_ref, orig_ref, boxes_ref, boxesT_ref, out_ref):
    H = added_ref.shape[2]
    W = added_ref.shape[3]

    pred = jnp.sum(added_ref[0], axis=0)
    orig = jnp.sum(orig_ref[0, 1:], axis=0)
    occ_p = jnp.abs(pred) > 0.0
    occ_o = jnp.abs(orig) > 0.0
    occ_and = jnp.logical_and(occ_p, occ_o)
    occ_or = jnp.logical_or(occ_p, occ_o)

    bT = boxesT_ref[0]
    bC = boxes_ref[0]
    M = bC.shape[0]

    c_r = jnp.cos(bT[6:7, :])
    s_r = jnp.sin(bT[6:7, :])
    iex_r = 2.0 / bT[3:4, :]
    iey_r = 2.0 / bT[4:5, :]
    a1 = c_r * iex_r
    a2 = -s_r * iey_r

    c_c = jnp.cos(bC[:, 6:7])
    s_c = jnp.sin(bC[:, 6:7])
    iex_c = 2.0 / bC[:, 3:4]
    iey_c = 2.0 / bC[:, 4:5]
    zok_c = jnp.abs(bC[:, 2:3]) < bC[:, 5:6] * 0.5
    tx_c = jnp.where(zok_c, -(bC[:, 0:1] * c_c + bC[:, 1:2] * s_c) * iex_c, 1e9)
    ty_c = jnp.where(zok_c, (bC[:, 0:1] * s_c - bC[:, 1:2] * c_c) * iey_c, 1e9)

    yrow = lax.broadcasted_iota(jnp.int32, (1, W), 1).astype(jnp.float32) * 0.8
    V1 = (s_c * iex_c) * yrow + tx_c
    V2 = (c_c * iey_c) * yrow + ty_c

    xmap = lax.broadcasted_iota(jnp.int32, (H, W), 0).astype(jnp.float32) * 0.8

    score = None
    for m in range(M):
        lx = xmap * a1[0:1, m : m + 1] + V1[m : m + 1, :]
        ly = xmap * a2[0:1, m : m + 1] + V2[m : m + 1, :]
        d = jnp.maximum(jnp.abs(lx), jnp.abs(ly))
        score = d if score is None else jnp.minimum(score, d)
    in_any = score < 1.0

    w_i = jnp.where(jnp.logical_and(in_any, occ_and), 1.0, 0.0)
    w_u = jnp.where(jnp.logical_and(in_any, occ_or), 1.0, 0.0)
    fi = jnp.zeros((8, 128), jnp.float32)
    fu = jnp.zeros((8, 128), jnp.float32)
    for r in range(H // 8):
        for cc in range(W // 128):
            fi = fi + w_i[8 * r : 8 * r + 8, 128 * cc : 128 * cc + 128]
            fu = fu + w_u[8 * r : 8 * r + 8, 128 * cc : 128 * cc + 128]

    inter = jnp.sum(fi)
    union = jnp.sum(fu)
    lane = lax.broadcasted_iota(jnp.int32, (1, 1, 128), 2)
    out_ref[...] = (jnp.where(lane == 0, inter, 0.0)
                    + jnp.where(lane == 1, union, 0.0))


def kernel(added_points, original_points, boxes):
    B, C, H, W = added_points.shape
    M = boxes.shape[1]
    boxesT = jnp.transpose(boxes, (0, 2, 1))

    out = pl.pallas_call(
        _body,
        grid=(B,),
        in_specs=[
            pl.BlockSpec((1, C, H, W), lambda b: (b, 0, 0, 0)),
            pl.BlockSpec((1, C + 1, H, W), lambda b: (b, 0, 0, 0)),
            pl.BlockSpec((1, M, 7), lambda b: (b, 0, 0)),
            pl.BlockSpec((1, 7, M), lambda b: (b, 0, 0)),
        ],
        out_specs=pl.BlockSpec((1, 1, 128), lambda b: (b, 0, 0)),
        out_shape=jax.ShapeDtypeStruct((B, 1, 128), jnp.float32),
        compiler_params=pltpu.CompilerParams(
            dimension_semantics=("arbitrary",)),
    )(added_points, original_points, boxes, boxesT)

    inter = out[:, 0, 0]
    union = out[:, 0, 1]
    return jnp.mean(M * inter / (union + 1e-6))
